# baseline (speedup 1.0000x reference)
import functools

import jax
import jax.numpy as jnp
from jax import lax
from jax.experimental import pallas as pl
from jax.experimental.pallas import tpu as pltpu
from jax.experimental.pallas import tpu_sc as plsc

B, C, N, V, T = 8, 1024, 16384, 4, 512
NSAMP, NH = 256, 8
K = NSAMP // V
HD = C // NH
S = NSAMP + T
NCH = 2048
NJ = N // NCH


NR = B * V
CH = 8
CL = N // CH
T0 = 24
BIG = N


def _topk_body(pw_ref, idx_ref, vw_ref):
    vw_ref[...] = pw_ref[...].reshape(NR, CH, CL)
    nio = (jax.lax.broadcasted_iota(jnp.int32, (NR, CH, CL), 2)
           + jax.lax.broadcasted_iota(jnp.int32, (NR, CH, CL), 1) * CL)
    tio = jax.lax.broadcasted_iota(jnp.int32, (NR, CH, T0), 2)

    def extract(t, carry):
        cval, cidx = carry
        cur = vw_ref[...]
        mx = jnp.max(cur, axis=2, keepdims=True)
        am = jnp.min(jnp.where(cur == mx, nio, BIG), axis=2, keepdims=True)
        vw_ref[...] = jnp.where(nio == am, jnp.float32(-1.0), cur)
        sel = tio == t
        return jnp.where(sel, mx, cval), jnp.where(sel, am, cidx)

    cval0 = jnp.full((NR, CH, T0), -1.0, jnp.float32)
    cidx0 = jnp.full((NR, CH, T0), BIG, jnp.int32)
    cval, cidx = jax.lax.fori_loop(0, T0, extract, (cval0, cidx0))

    kio = jax.lax.broadcasted_iota(jnp.int32, (NR, K), 1)

    def merge(t, carry):
        cv, ci, acc, used = carry
        mx = jnp.max(jnp.max(cv, axis=2, keepdims=True), axis=1, keepdims=True)
        am = jnp.min(jnp.min(jnp.where(cv == mx, ci, BIG), axis=2,
                             keepdims=True), axis=1, keepdims=True)
        hit = ci == am
        cv = jnp.where(hit, jnp.float32(-1.0), cv)
        used = used + hit.astype(jnp.int32)
        return cv, ci, jnp.where(kio == t, am[:, 0, :], acc), used

    acc0 = jnp.zeros((NR, K), jnp.int32)
    used0 = jnp.zeros((NR, CH, T0), jnp.int32)
    cval, cidx, acc, used = jax.lax.fori_loop(
        0, K, merge, (cval, cidx, acc0, used0))
    idx_ref[...] = acc.reshape(B, V, K)

    bad = jnp.max(jnp.max(jnp.sum(used, axis=2), axis=1), axis=0) >= T0

    @pl.when(bad)
    def _():
        vw_ref[...] = pw_ref[...].reshape(NR, CH, CL)

        def full(t, acc):
            cur = vw_ref[...]
            mx = jnp.max(jnp.max(cur, axis=2, keepdims=True), axis=1,
                         keepdims=True)
            am = jnp.min(jnp.min(jnp.where(cur == mx, nio, BIG), axis=2,
                                 keepdims=True), axis=1, keepdims=True)
            vw_ref[...] = jnp.where(nio == am, jnp.float32(-1.0), cur)
            return jnp.where(kio == t, am[:, 0, :], acc)

        idx_ref[...] = jax.lax.fori_loop(
            0, K, full, jnp.zeros((NR, K), jnp.int32)).reshape(B, V, K)


NW_SC = 32
GW = 128
INNER = 16
OUTER = (B * NSAMP * C) // (NW_SC * INNER * GW)


def _sc_gather(table, addr):
    mesh = plsc.VectorSubcoreMesh(core_axis_name="c", subcore_axis_name="s")

    @functools.partial(
        pl.kernel, mesh=mesh,
        out_type=jax.ShapeDtypeStruct((NW_SC, OUTER, INNER, GW), jnp.float32),
        scratch_types=[pltpu.VMEM((INNER, GW), jnp.int32),
                       pltpu.VMEM((INNER, GW), jnp.float32),
                       pltpu.SemaphoreType.DMA],
    )
    def run(table_hbm, addr_hbm, out_hbm, idx_v, val_v, sem):
        w = lax.axis_index("s") * 2 + lax.axis_index("c")

        def round_(g, carry):
            pltpu.sync_copy(addr_hbm.at[w, g], idx_v)
            handles = [
                pltpu.async_copy(table_hbm.at[idx_v.at[j]], val_v.at[j], sem)
                for j in range(INNER)]
            for h in handles:
                h.wait()
            pltpu.sync_copy(val_v, out_hbm.at[w, g])
            return carry

        lax.fori_loop(0, OUTER, round_, 0)

    return run(table, addr)


def _gather_body(idx_ref, pf_ref, out_ref):
    j = pl.program_id(1)
    idxc = idx_ref[0]
    ni = jax.lax.broadcasted_iota(jnp.int32, (NSAMP, NCH), 1) + j * NCH
    oh = (ni == idxc).astype(jnp.bfloat16)
    pfb = pf_ref[0].astype(jnp.bfloat16)
    part = jax.lax.dot_general(oh, pfb, (((1,), (1,)), ((), ())),
                               preferred_element_type=jnp.float32)

    @pl.when(j == 0)
    def _():
        out_ref[0] = part

    @pl.when(j > 0)
    def _():
        out_ref[0] += part


def _qkv_body(x_ref, w_ref, b_ref, out_ref):
    acc = jax.lax.dot_general(x_ref[0], w_ref[...], (((1,), (1,)), ((), ())),
                              preferred_element_type=jnp.float32)
    out_ref[0] = (acc + b_ref[...]).astype(jnp.bfloat16)


def _attn_body(q_ref, k_ref, v_ref, bias_ref, wo_ref, bo_ref, out_ref, oacc_ref):
    h = pl.program_id(1)
    s = jax.lax.dot_general(q_ref[0], k_ref[0], (((1,), (1,)), ((), ())),
                            preferred_element_type=jnp.float32)
    s = s * jnp.float32(1.0 / 128 ** 0.5) + bias_ref[0]
    e = jnp.exp(s)
    r = 1.0 / jnp.sum(e, axis=1, keepdims=True)
    o = jax.lax.dot_general(e.astype(jnp.bfloat16), v_ref[0],
                            (((1,), (0,)), ((), ())),
                            preferred_element_type=jnp.float32)
    oacc_ref[:, pl.ds(h * HD, HD)] = (o * r).astype(jnp.bfloat16)

    @pl.when(h == NH - 1)
    def _():
        out_ref[0] = jax.lax.dot_general(
            oacc_ref[...], wo_ref[...], (((1,), (1,)), ((), ())),
            preferred_element_type=jnp.float32) + bo_ref[...]


def kernel(point_features, point_masks, t_feat, t_mask, xyz, W_in, b_in, W_out, b_out):
    xyz_t = xyz.transpose(0, 2, 1)
    masked_xyz = xyz_t[:, None, :, :] * point_masks[..., None]
    valid = jnp.clip(point_masks.sum(axis=-1, keepdims=True), 1.0, None)
    center = masked_xyz.sum(axis=-2) / valid
    d2 = ((xyz_t[:, :, None, :] - center[:, None, :, :]) ** 2).sum(-1)
    dist = jnp.sqrt(jnp.clip(d2, 1e-12, None)).transpose(0, 2, 1)
    prob = jnp.exp(-dist)
    voting_ratio = valid.squeeze(-1) / N
    vote_weight = jnp.einsum('bi,bij->bj', voting_ratio, point_masks)
    vote_weight = vote_weight[:, None, :] * prob
    point_weight = jax.nn.softmax(vote_weight, axis=-1)

    idx = pl.pallas_call(
        _topk_body,
        in_specs=[pl.BlockSpec((B, V, CH, CL), lambda: (0, 0, 0, 0))],
        out_specs=pl.BlockSpec((B, V, K), lambda: (0, 0, 0)),
        out_shape=jax.ShapeDtypeStruct((B, V, K), jnp.int32),
        scratch_shapes=[pltpu.VMEM((NR, CH, CL), jnp.float32)],
    )(point_weight.reshape(B, V, CH, CL))

    addr = ((jnp.arange(B, dtype=jnp.int32)[:, None, None] * C
             + jnp.arange(C, dtype=jnp.int32)[None, None, :]) * N
            + idx.reshape(B, NSAMP)[:, :, None])
    sampled = _sc_gather(
        point_features.reshape(B * C * N),
        addr.reshape(NW_SC, OUTER, INNER, GW)).reshape(B, NSAMP, C)

    combined = jnp.concatenate([sampled, t_feat], axis=1).astype(jnp.bfloat16)

    qkv = pl.pallas_call(
        _qkv_body,
        grid=(B, 4),
        in_specs=[pl.BlockSpec((1, S, C), lambda b, j: (b, 0, 0)),
                  pl.BlockSpec((S, C), lambda b, j: (j, 0)),
                  pl.BlockSpec((1, S), lambda b, j: (0, j))],
        out_specs=pl.BlockSpec((1, S, S), lambda b, j: (b, 0, j)),
        out_shape=jax.ShapeDtypeStruct((B, S, 3 * C), jnp.bfloat16),
        compiler_params=pltpu.CompilerParams(
            dimension_semantics=("parallel", "arbitrary")),
    )(combined, W_in.astype(jnp.bfloat16), b_in.reshape(1, 3 * C))

    biasf = jnp.concatenate(
        [jnp.zeros((B, NSAMP), jnp.float32),
         jnp.where(t_mask, jnp.float32(0), jnp.float32(-1e9))],
        axis=1).reshape(B, 1, S)

    out = pl.pallas_call(
        _attn_body,
        grid=(B, NH),
        in_specs=[pl.BlockSpec((1, S, HD), lambda b, h: (b, 0, h)),
                  pl.BlockSpec((1, S, HD), lambda b, h: (b, 0, NH + h)),
                  pl.BlockSpec((1, S, HD), lambda b, h: (b, 0, 2 * NH + h)),
                  pl.BlockSpec((1, 1, S), lambda b, h: (b, 0, 0)),
                  pl.BlockSpec((C, C), lambda b, h: (0, 0)),
                  pl.BlockSpec((1, C), lambda b, h: (0, 0))],
        out_specs=pl.BlockSpec((1, S, C), lambda b, h: (b, 0, 0)),
        out_shape=jax.ShapeDtypeStruct((B, S, C), jnp.float32),
        scratch_shapes=[pltpu.VMEM((S, C), jnp.bfloat16)],
        compiler_params=pltpu.CompilerParams(
            dimension_semantics=("parallel", "arbitrary")),
    )(qkv, qkv, qkv, biasf, W_out.astype(jnp.bfloat16), b_out.reshape(1, C))

    combined_mask = jnp.concatenate(
        [jnp.ones((B, NSAMP), dtype=jnp.bool_), t_mask], axis=1)
    return out, combined_mask


# speedup vs baseline: 1.7307x; 1.7307x over previous
import jax
import jax.numpy as jnp
from jax.experimental import pallas as pl
from jax.experimental.pallas import tpu as pltpu

B, C, N, V, T = 8, 1024, 16384, 4, 512
NSAMP, NH = 256, 8
K = NSAMP // V
HD = C // NH
S = NSAMP + T
NCH = 4096
NJ = N // NCH


NR = B * V
CH = 8
CL = N // CH
T0 = 24
BIG = N


def _topk_body(pw_ref, idx_ref, vw_ref):
    vw_ref[...] = pw_ref[...].reshape(NR, CH, CL)
    nio = (jax.lax.broadcasted_iota(jnp.int32, (NR, CH, CL), 2)
           + jax.lax.broadcasted_iota(jnp.int32, (NR, CH, CL), 1) * CL)
    tio = jax.lax.broadcasted_iota(jnp.int32, (NR, CH, T0), 2)

    def extract(t, carry):
        cval, cidx = carry
        cur = vw_ref[...]
        mx = jnp.max(cur, axis=2, keepdims=True)
        am = jnp.min(jnp.where(cur == mx, nio, BIG), axis=2, keepdims=True)
        vw_ref[...] = jnp.where(nio == am, jnp.float32(-1.0), cur)
        sel = tio == t
        return jnp.where(sel, mx, cval), jnp.where(sel, am, cidx)

    cval0 = jnp.full((NR, CH, T0), -1.0, jnp.float32)
    cidx0 = jnp.full((NR, CH, T0), BIG, jnp.int32)
    cval, cidx = jax.lax.fori_loop(0, T0, extract, (cval0, cidx0))

    kio = jax.lax.broadcasted_iota(jnp.int32, (NR, K), 1)

    def merge(t, carry):
        cv, ci, acc, used = carry
        mx = jnp.max(jnp.max(cv, axis=2, keepdims=True), axis=1, keepdims=True)
        am = jnp.min(jnp.min(jnp.where(cv == mx, ci, BIG), axis=2,
                             keepdims=True), axis=1, keepdims=True)
        hit = ci == am
        cv = jnp.where(hit, jnp.float32(-1.0), cv)
        used = used + hit.astype(jnp.int32)
        return cv, ci, jnp.where(kio == t, am[:, 0, :], acc), used

    acc0 = jnp.zeros((NR, K), jnp.int32)
    used0 = jnp.zeros((NR, CH, T0), jnp.int32)
    cval, cidx, acc, used = jax.lax.fori_loop(
        0, K, merge, (cval, cidx, acc0, used0))
    idx_ref[...] = acc.reshape(B, V, K)

    bad = jnp.max(jnp.max(jnp.sum(used, axis=2), axis=1), axis=0) >= T0

    @pl.when(bad)
    def _():
        vw_ref[...] = pw_ref[...].reshape(NR, CH, CL)

        def full(t, acc):
            cur = vw_ref[...]
            mx = jnp.max(jnp.max(cur, axis=2, keepdims=True), axis=1,
                         keepdims=True)
            am = jnp.min(jnp.min(jnp.where(cur == mx, nio, BIG), axis=2,
                                 keepdims=True), axis=1, keepdims=True)
            vw_ref[...] = jnp.where(nio == am, jnp.float32(-1.0), cur)
            return jnp.where(kio == t, am[:, 0, :], acc)

        idx_ref[...] = jax.lax.fori_loop(
            0, K, full, jnp.zeros((NR, K), jnp.int32)).reshape(B, V, K)


def _gather_body(idx_ref, pf_ref, out_ref):
    j = pl.program_id(1)
    idxc = idx_ref[0]
    ni = jax.lax.broadcasted_iota(jnp.int32, (NSAMP, NCH), 1) + j * NCH
    oh = (ni == idxc).astype(jnp.bfloat16)
    pfb = pf_ref[0].astype(jnp.bfloat16)
    part = jax.lax.dot_general(oh, pfb, (((1,), (1,)), ((), ())),
                               preferred_element_type=jnp.float32)

    @pl.when(j == 0)
    def _():
        out_ref[0] = part

    @pl.when(j > 0)
    def _():
        out_ref[0] += part


def _qkv_body(x_ref, w_ref, b_ref, out_ref):
    acc = jax.lax.dot_general(x_ref[0], w_ref[...], (((1,), (1,)), ((), ())),
                              preferred_element_type=jnp.float32)
    out_ref[0] = (acc + b_ref[...]).astype(jnp.bfloat16)


def _attn_body(q_ref, k_ref, v_ref, bias_ref, wo_ref, bo_ref, out_ref, oacc_ref):
    h = pl.program_id(1)
    s = jax.lax.dot_general(q_ref[0], k_ref[0], (((1,), (1,)), ((), ())),
                            preferred_element_type=jnp.float32)
    s = s * jnp.float32(1.0 / 128 ** 0.5) + bias_ref[0]
    e = jnp.exp(s).astype(jnp.bfloat16)
    r = 1.0 / jnp.sum(e.astype(jnp.float32), axis=1, keepdims=True)
    o = jax.lax.dot_general(e, v_ref[0],
                            (((1,), (0,)), ((), ())),
                            preferred_element_type=jnp.float32)
    oacc_ref[:, pl.ds(h * HD, HD)] = (o * r).astype(jnp.bfloat16)

    @pl.when(h == NH - 1)
    def _():
        out_ref[0] = jax.lax.dot_general(
            oacc_ref[...], wo_ref[...], (((1,), (1,)), ((), ())),
            preferred_element_type=jnp.float32) + bo_ref[...]


def kernel(point_features, point_masks, t_feat, t_mask, xyz, W_in, b_in, W_out, b_out):
    xyz_t = xyz.transpose(0, 2, 1)
    masked_xyz = xyz_t[:, None, :, :] * point_masks[..., None]
    valid = jnp.clip(point_masks.sum(axis=-1, keepdims=True), 1.0, None)
    center = masked_xyz.sum(axis=-2) / valid
    d2 = ((xyz_t[:, :, None, :] - center[:, None, :, :]) ** 2).sum(-1)
    dist = jnp.sqrt(jnp.clip(d2, 1e-12, None)).transpose(0, 2, 1)
    prob = jnp.exp(-dist)
    voting_ratio = valid.squeeze(-1) / N
    vote_weight = jnp.einsum('bi,bij->bj', voting_ratio, point_masks)
    vote_weight = vote_weight[:, None, :] * prob
    point_weight = jax.nn.softmax(vote_weight, axis=-1)

    idx = pl.pallas_call(
        _topk_body,
        in_specs=[pl.BlockSpec((B, V, CH, CL), lambda: (0, 0, 0, 0))],
        out_specs=pl.BlockSpec((B, V, K), lambda: (0, 0, 0)),
        out_shape=jax.ShapeDtypeStruct((B, V, K), jnp.int32),
        scratch_shapes=[pltpu.VMEM((NR, CH, CL), jnp.float32)],
    )(point_weight.reshape(B, V, CH, CL))

    idxc = idx.reshape(B, NSAMP, 1)

    sampled = pl.pallas_call(
        _gather_body,
        grid=(B, NJ),
        in_specs=[pl.BlockSpec((1, NSAMP, 1), lambda b, j: (b, 0, 0)),
                  pl.BlockSpec((1, C, NCH), lambda b, j: (b, 0, j))],
        out_specs=pl.BlockSpec((1, NSAMP, C), lambda b, j: (b, 0, 0)),
        out_shape=jax.ShapeDtypeStruct((B, NSAMP, C), jnp.float32),
        compiler_params=pltpu.CompilerParams(
            dimension_semantics=("parallel", "arbitrary")),
    )(idxc, point_features)

    combined = jnp.concatenate([sampled, t_feat], axis=1).astype(jnp.bfloat16)

    qkv = pl.pallas_call(
        _qkv_body,
        grid=(B, 4),
        in_specs=[pl.BlockSpec((1, S, C), lambda b, j: (b, 0, 0)),
                  pl.BlockSpec((S, C), lambda b, j: (j, 0)),
                  pl.BlockSpec((1, S), lambda b, j: (0, j))],
        out_specs=pl.BlockSpec((1, S, S), lambda b, j: (b, 0, j)),
        out_shape=jax.ShapeDtypeStruct((B, S, 3 * C), jnp.bfloat16),
        compiler_params=pltpu.CompilerParams(
            dimension_semantics=("parallel", "arbitrary")),
    )(combined, W_in.astype(jnp.bfloat16), b_in.reshape(1, 3 * C))

    biasf = jnp.concatenate(
        [jnp.zeros((B, NSAMP), jnp.float32),
         jnp.where(t_mask, jnp.float32(0), jnp.float32(-1e9))],
        axis=1).reshape(B, 1, S)

    out = pl.pallas_call(
        _attn_body,
        grid=(B, NH),
        in_specs=[pl.BlockSpec((1, S, HD), lambda b, h: (b, 0, h)),
                  pl.BlockSpec((1, S, HD), lambda b, h: (b, 0, NH + h)),
                  pl.BlockSpec((1, S, HD), lambda b, h: (b, 0, 2 * NH + h)),
                  pl.BlockSpec((1, 1, S), lambda b, h: (b, 0, 0)),
                  pl.BlockSpec((C, C), lambda b, h: (0, 0)),
                  pl.BlockSpec((1, C), lambda b, h: (0, 0))],
        out_specs=pl.BlockSpec((1, S, C), lambda b, h: (b, 0, 0)),
        out_shape=jax.ShapeDtypeStruct((B, S, C), jnp.float32),
        scratch_shapes=[pltpu.VMEM((S, C), jnp.bfloat16)],
        compiler_params=pltpu.CompilerParams(
            dimension_semantics=("parallel", "arbitrary")),
    )(qkv, qkv, qkv, biasf, W_out.astype(jnp.bfloat16), b_out.reshape(1, C))

    combined_mask = jnp.concatenate(
        [jnp.ones((B, NSAMP), dtype=jnp.bool_), t_mask], axis=1)
    return out, combined_mask


# speedup vs baseline: 1.7404x; 1.0056x over previous
import jax
import jax.numpy as jnp
from jax.experimental import pallas as pl
from jax.experimental.pallas import tpu as pltpu

B, C, N, V, T = 8, 1024, 16384, 4, 512
NSAMP, NH = 256, 8
K = NSAMP // V
HD = C // NH
S = NSAMP + T
NCH = 2048
NJ = N // NCH


NR = B * V
CH = 8
CL = N // CH
T0 = 24
BIG = N


def _topk_body(pw_ref, idx_ref, vw_ref):
    vw_ref[...] = pw_ref[...].reshape(NR, CH, CL)
    nio = (jax.lax.broadcasted_iota(jnp.int32, (NR, CH, CL), 2)
           + jax.lax.broadcasted_iota(jnp.int32, (NR, CH, CL), 1) * CL)
    tio = jax.lax.broadcasted_iota(jnp.int32, (NR, CH, T0), 2)

    def extract(t, carry):
        cval, cidx = carry
        cur = vw_ref[...]
        mx = jnp.max(cur, axis=2, keepdims=True)
        am = jnp.min(jnp.where(cur == mx, nio, BIG), axis=2, keepdims=True)
        vw_ref[...] = jnp.where(nio == am, jnp.float32(-1.0), cur)
        sel = tio == t
        return jnp.where(sel, mx, cval), jnp.where(sel, am, cidx)

    cval0 = jnp.full((NR, CH, T0), -1.0, jnp.float32)
    cidx0 = jnp.full((NR, CH, T0), BIG, jnp.int32)
    cval, cidx = jax.lax.fori_loop(0, T0, extract, (cval0, cidx0))

    kio = jax.lax.broadcasted_iota(jnp.int32, (NR, K), 1)

    def merge(t, carry):
        cv, ci, acc, used = carry
        mx = jnp.max(jnp.max(cv, axis=2, keepdims=True), axis=1, keepdims=True)
        am = jnp.min(jnp.min(jnp.where(cv == mx, ci, BIG), axis=2,
                             keepdims=True), axis=1, keepdims=True)
        hit = ci == am
        cv = jnp.where(hit, jnp.float32(-1.0), cv)
        used = used + hit.astype(jnp.int32)
        return cv, ci, jnp.where(kio == t, am[:, 0, :], acc), used

    acc0 = jnp.zeros((NR, K), jnp.int32)
    used0 = jnp.zeros((NR, CH, T0), jnp.int32)
    cval, cidx, acc, used = jax.lax.fori_loop(
        0, K, merge, (cval, cidx, acc0, used0))
    idx_ref[...] = acc.reshape(B, V, K)

    bad = jnp.max(jnp.max(jnp.sum(used, axis=2), axis=1), axis=0) >= T0

    @pl.when(bad)
    def _():
        vw_ref[...] = pw_ref[...].reshape(NR, CH, CL)

        def full(t, acc):
            cur = vw_ref[...]
            mx = jnp.max(jnp.max(cur, axis=2, keepdims=True), axis=1,
                         keepdims=True)
            am = jnp.min(jnp.min(jnp.where(cur == mx, nio, BIG), axis=2,
                                 keepdims=True), axis=1, keepdims=True)
            vw_ref[...] = jnp.where(nio == am, jnp.float32(-1.0), cur)
            return jnp.where(kio == t, am[:, 0, :], acc)

        idx_ref[...] = jax.lax.fori_loop(
            0, K, full, jnp.zeros((NR, K), jnp.int32)).reshape(B, V, K)


def _gather_body(idx_ref, pf_ref, out_ref):
    j = pl.program_id(1)
    idxc = idx_ref[0]
    ni = jax.lax.broadcasted_iota(jnp.int32, (NSAMP, NCH), 1) + j * NCH
    oh = (ni == idxc).astype(jnp.bfloat16)
    pfb = pf_ref[0].astype(jnp.bfloat16)
    part = jax.lax.dot_general(oh, pfb, (((1,), (1,)), ((), ())),
                               preferred_element_type=jnp.float32)

    @pl.when(j == 0)
    def _():
        out_ref[0] = part

    @pl.when(j > 0)
    def _():
        out_ref[0] += part


def _qkv_body(x_ref, w_ref, b_ref, out_ref):
    acc = jax.lax.dot_general(x_ref[0], w_ref[...], (((1,), (1,)), ((), ())),
                              preferred_element_type=jnp.float32)
    out_ref[0] = (acc + b_ref[...]).astype(jnp.bfloat16)


def _attn_body(q_ref, k_ref, v_ref, bias_ref, wo_ref, bo_ref, out_ref, oacc_ref):
    h = pl.program_id(1)
    s = jax.lax.dot_general(q_ref[0], k_ref[0], (((1,), (1,)), ((), ())),
                            preferred_element_type=jnp.float32)
    s = s * jnp.float32(1.0 / 128 ** 0.5) + bias_ref[0]
    e = jnp.exp(s)
    r = 1.0 / jnp.sum(e, axis=1, keepdims=True)
    o = jax.lax.dot_general(e.astype(jnp.bfloat16), v_ref[0],
                            (((1,), (0,)), ((), ())),
                            preferred_element_type=jnp.float32)
    oacc_ref[:, pl.ds(h * HD, HD)] = (o * r).astype(jnp.bfloat16)

    @pl.when(h == NH - 1)
    def _():
        out_ref[0] = jax.lax.dot_general(
            oacc_ref[...], wo_ref[...], (((1,), (1,)), ((), ())),
            preferred_element_type=jnp.float32) + bo_ref[...]


def kernel(point_features, point_masks, t_feat, t_mask, xyz, W_in, b_in, W_out, b_out):
    xyz_t = xyz.transpose(0, 2, 1)
    masked_xyz = xyz_t[:, None, :, :] * point_masks[..., None]
    valid = jnp.clip(point_masks.sum(axis=-1, keepdims=True), 1.0, None)
    center = masked_xyz.sum(axis=-2) / valid
    d2 = ((xyz_t[:, :, None, :] - center[:, None, :, :]) ** 2).sum(-1)
    dist = jnp.sqrt(jnp.clip(d2, 1e-12, None)).transpose(0, 2, 1)
    prob = jnp.exp(-dist)
    voting_ratio = valid.squeeze(-1) / N
    vote_weight = jnp.einsum('bi,bij->bj', voting_ratio, point_masks)
    vote_weight = vote_weight[:, None, :] * prob
    point_weight = jax.nn.softmax(vote_weight, axis=-1)

    idx = pl.pallas_call(
        _topk_body,
        in_specs=[pl.BlockSpec((B, V, CH, CL), lambda: (0, 0, 0, 0))],
        out_specs=pl.BlockSpec((B, V, K), lambda: (0, 0, 0)),
        out_shape=jax.ShapeDtypeStruct((B, V, K), jnp.int32),
        scratch_shapes=[pltpu.VMEM((NR, CH, CL), jnp.float32)],
    )(point_weight.reshape(B, V, CH, CL))

    idxc = idx.reshape(B, NSAMP, 1)

    sampled = pl.pallas_call(
        _gather_body,
        grid=(B, NJ),
        in_specs=[pl.BlockSpec((1, NSAMP, 1), lambda b, j: (b, 0, 0)),
                  pl.BlockSpec((1, C, NCH), lambda b, j: (b, 0, j))],
        out_specs=pl.BlockSpec((1, NSAMP, C), lambda b, j: (b, 0, 0)),
        out_shape=jax.ShapeDtypeStruct((B, NSAMP, C), jnp.float32),
        compiler_params=pltpu.CompilerParams(
            dimension_semantics=("parallel", "arbitrary")),
    )(idxc, point_features)

    combined = jnp.concatenate([sampled, t_feat], axis=1).astype(jnp.bfloat16)

    qkv = pl.pallas_call(
        _qkv_body,
        grid=(B, 4),
        in_specs=[pl.BlockSpec((1, S, C), lambda b, j: (b, 0, 0)),
                  pl.BlockSpec((S, C), lambda b, j: (j, 0)),
                  pl.BlockSpec((1, S), lambda b, j: (0, j))],
        out_specs=pl.BlockSpec((1, S, S), lambda b, j: (b, 0, j)),
        out_shape=jax.ShapeDtypeStruct((B, S, 3 * C), jnp.bfloat16),
        compiler_params=pltpu.CompilerParams(
            dimension_semantics=("parallel", "arbitrary")),
    )(combined, W_in.astype(jnp.bfloat16), b_in.reshape(1, 3 * C))

    biasf = jnp.concatenate(
        [jnp.zeros((B, NSAMP), jnp.float32),
         jnp.where(t_mask, jnp.float32(0), jnp.float32(-1e9))],
        axis=1).reshape(B, 1, S)

    out = pl.pallas_call(
        _attn_body,
        grid=(B, NH),
        in_specs=[pl.BlockSpec((1, S, HD), lambda b, h: (b, 0, h)),
                  pl.BlockSpec((1, S, HD), lambda b, h: (b, 0, NH + h)),
                  pl.BlockSpec((1, S, HD), lambda b, h: (b, 0, 2 * NH + h)),
                  pl.BlockSpec((1, 1, S), lambda b, h: (b, 0, 0)),
                  pl.BlockSpec((C, C), lambda b, h: (0, 0)),
                  pl.BlockSpec((1, C), lambda b, h: (0, 0))],
        out_specs=pl.BlockSpec((1, S, C), lambda b, h: (b, 0, 0)),
        out_shape=jax.ShapeDtypeStruct((B, S, C), jnp.float32),
        scratch_shapes=[pltpu.VMEM((S, C), jnp.bfloat16)],
        compiler_params=pltpu.CompilerParams(
            dimension_semantics=("parallel", "arbitrary")),
    )(qkv, qkv, qkv, biasf, W_out.astype(jnp.bfloat16), b_out.reshape(1, C))

    combined_mask = jnp.concatenate(
        [jnp.ones((B, NSAMP), dtype=jnp.bool_), t_mask], axis=1)
    return out, combined_mask
